# baseline (speedup 1.0000x reference)
import jax
import jax.numpy as jnp
from jax.experimental import pallas as pl
from jax.experimental.pallas import tpu as pltpu

_CHUNK = 1024
_NBUF = 4


def kernel(x, emb_weight):
    sl = x.shape[1]
    dim = emb_weight.shape[1]
    nchunks = sl // _CHUNK

    def body(in_hbm, out_hbm, buf, isems, osems):
        def load(i):
            b = i % _NBUF
            return pltpu.make_async_copy(
                in_hbm.at[pl.ds(i * _CHUNK, _CHUNK)], buf.at[b], isems.at[b]
            )

        def store(i):
            b = i % _NBUF
            return pltpu.make_async_copy(
                buf.at[b], out_hbm.at[pl.ds(i * _CHUNK, _CHUNK)], osems.at[b]
            )

        for i in range(min(_NBUF, nchunks)):
            load(i).start()
        for i in range(nchunks):
            if i >= _NBUF:
                store(i - _NBUF).wait()
                load(i).start()
            load(i).wait()
            store(i).start()
        for i in range(max(0, nchunks - _NBUF), nchunks):
            store(i).wait()

    return pl.pallas_call(
        body,
        out_shape=jax.ShapeDtypeStruct((sl, dim), emb_weight.dtype),
        in_specs=[pl.BlockSpec(memory_space=pl.ANY)],
        out_specs=pl.BlockSpec(memory_space=pl.ANY),
        scratch_shapes=[
            pltpu.VMEM((_NBUF, _CHUNK, dim), jnp.float32),
            pltpu.SemaphoreType.DMA((_NBUF,)),
            pltpu.SemaphoreType.DMA((_NBUF,)),
        ],
    )(emb_weight)
